# baseline (speedup 1.0000x reference)
import functools
import jax
import jax.numpy as jnp
from jax.experimental import pallas as pl

_EPS = 1e-5
_NG = 64
_F = 1024


def _mm_block(a_ref, b_ref, o_ref):
    o_ref[...] = jnp.dot(a_ref[...], b_ref[...],
                         preferred_element_type=jnp.float32)


def _matmul(a, b, bm, bn):
    m, k = a.shape
    k2, n = b.shape
    grid = (m // bm, n // bn)
    return pl.pallas_call(
        _mm_block,
        grid=grid,
        in_specs=[
            pl.BlockSpec((bm, k), lambda i, j: (i, 0)),
            pl.BlockSpec((k, bn), lambda i, j: (0, j)),
        ],
        out_specs=pl.BlockSpec((bm, bn), lambda i, j: (i, j)),
        out_shape=jax.ShapeDtypeStruct((m, n), jnp.float32),
    )(a, b)


def kernel(x, edge_index, batch, W0, b0, g0, be0, W1, b1, g1, be1,
           W2, b2, g2, be2, Wr, br):
    n = x.shape[0]
    E = edge_index.shape[1]
    src, dst = edge_index[0], edge_index[1]

    Wcat = jnp.concatenate([
        W0,
        W1[:_F] - W1[_F:], W1[_F:],
        W2[:_F] - W2[_F:], W2[_F:],
    ], axis=1)
    H = _matmul(x, Wcat, 400, 512)
    H0 = H[:, 0 * _F:1 * _F]
    A1 = H[:, 1 * _F:2 * _F]
    B1 = H[:, 2 * _F:3 * _F]
    A2 = H[:, 3 * _F:4 * _F]
    B2 = H[:, 4 * _F:5 * _F]

    cnt_dst = jax.ops.segment_sum(jnp.ones((E,), jnp.float32), dst,
                                  num_segments=n)
    cnt_src = jax.ops.segment_sum(jnp.ones((E,), jnp.float32), src,
                                  num_segments=n)

    P = (batch[None, :] == jnp.arange(_NG)[:, None]).astype(jnp.float32)
    Pn = P / jnp.maximum(P.sum(1, keepdims=True), 1.0)

    mu0 = jnp.mean(H0, axis=0) + b0
    var0 = jnp.mean((H0 + b0[None, :] - mu0[None, :]) ** 2, axis=0)
    s0 = g0 * jax.lax.rsqrt(var0 + _EPS)
    t0 = (b0 - mu0) * s0 + be0
    p0 = (Pn @ H0) * s0[None, :] + t0[None, :]

    def edgeconv(A, B, b, g, be):
        sig = jnp.sign(g)
        S = jax.ops.segment_sum(B[src], dst, num_segments=n)
        M = jax.ops.segment_max(B[src] * sig[None, :], dst, num_segments=n)
        sumA = cnt_dst @ A
        sumB = cnt_src @ B
        sumA2 = cnt_dst @ (A * A)
        sumB2 = cnt_src @ (B * B)
        cross = jnp.sum(A * S, axis=0)
        mu_nb = (sumA + sumB) / E
        var = (sumA2 + sumB2 + 2.0 * cross) / E - mu_nb ** 2
        s = g * jax.lax.rsqrt(var + _EPS)
        t = (b - mu_nb) * s + be
        mask = (cnt_dst > 0)[:, None]
        Mz = jnp.where(mask, M, 0.0)
        agg = A * s[None, :] + t[None, :] + jnp.abs(s)[None, :] * Mz
        agg = jnp.where(mask & jnp.isfinite(agg), agg, 0.0)
        return Pn @ agg

    p1 = edgeconv(A1, B1, b1, g1, be1)
    p2 = edgeconv(A2, B2, b2, g2, be2)
    acc = p0 + p1 + p2
    return acc @ Wr + br
